# Initial kernel scaffold; baseline (speedup 1.0000x reference)
import jax
import jax.numpy as jnp
from jax.experimental import pallas as pl


def kernel(x, edge_index, pos, W1m, b1m, W1a, b1a, W2m, b2m, W2a, b2a):
    raise NotImplementedError("write your pallas kernel here")


# speedup vs baseline: 9.3975x; 9.3975x over previous
import functools

import jax
import jax.numpy as jnp
from jax import lax
from jax.experimental import pallas as pl
from jax.experimental.pallas import tpu as pltpu
from jax.experimental.pallas import tpu_sc as plsc

N_NODES = 10000
N_EDGES = 320000
D_FEAT = 128
D_POS = 3
D_CTX = 32

NC = 2
NS = 16
NW = NC * NS
EPW = N_EDGES // NW
CH = 80
NCHUNK = EPW // CH
DG = 144
NACC = 10240
RPT = NACC // NS


def _sc_segment_sums(G, zeros, idx_i, idx_j):
  mesh = plsc.VectorSubcoreMesh(core_axis_name="c", subcore_axis_name="s")

  @functools.partial(
      pl.kernel,
      out_type=jax.ShapeDtypeStruct((NC, NACC, DG), jnp.float32),
      mesh=mesh,
      scratch_types=[
          pltpu.VMEM((CH,), jnp.int32),
          pltpu.VMEM((CH,), jnp.int32),
          pltpu.VMEM((CH, DG), jnp.float32),
          pltpu.VMEM_SHARED((NACC, DG), jnp.float32),
          pltpu.SemaphoreType.DMA,
      ],
      compiler_params=pltpu.CompilerParams(use_tc_tiling_on_sc=False),
  )
  def k(g_hbm, z_hbm, ii_hbm, jj_hbm, out_hbm, ibuf, jbuf, rows, acc, sem):
    c = lax.axis_index("c")
    s = lax.axis_index("s")
    wid = c * NS + s
    pltpu.sync_copy(z_hbm.at[pl.ds(s * RPT, RPT)], acc.at[pl.ds(s * RPT, RPT)])
    plsc.subcore_barrier()

    def body(kk, carry):
      base = pl.multiple_of(wid * EPW + kk * CH, CH)
      pltpu.sync_copy(jj_hbm.at[pl.ds(base, CH)], jbuf)
      pltpu.sync_copy(ii_hbm.at[pl.ds(base, CH)], ibuf)
      pltpu.async_copy(g_hbm.at[jbuf], rows, sem).wait()
      pltpu.sync_copy(rows, acc.at[ibuf], add=True)
      return carry

    lax.fori_loop(0, NCHUNK, body, 0)
    plsc.subcore_barrier()
    pltpu.sync_copy(acc.at[pl.ds(s * RPT, RPT)],
                    out_hbm.at[c, pl.ds(s * RPT, RPT)])

  return k(G, zeros, idx_i, idx_j)


def _epilogue_body(acc_ref, x_ref, pos_ref, w1m_ref, b1m_ref, w1a_ref,
                   b1a_ref, wa_ref, wb_ref, wc_ref, wd_ref, b2m_ref,
                   wax_ref, wag_ref, b2a_ref, out_ref):
  a = acc_ref[0] + acc_ref[1]
  Sx = a[:, :D_FEAT]
  Sp = a[:, D_FEAT:D_FEAT + D_POS]
  cnt = a[:, D_FEAT + D_POS:D_FEAT + D_POS + 1]
  has = (cnt > 0.0).astype(jnp.float32)
  inv = 1.0 / jnp.maximum(cnt, 1.0)
  xb = x_ref[...]
  Xm = Sx * inv
  Pm = Sp * inv - has * pos_ref[...]
  hi = lax.Precision.HIGHEST
  aggr1 = jnp.dot(Pm, w1m_ref[...], precision=hi) + has * b1m_ref[...]
  ctx = jnp.dot(aggr1, w1a_ref[...], precision=hi) + b1a_ref[...]
  aggr2 = (has * (jnp.dot(xb, wa_ref[...] - wb_ref[...], precision=hi)
                  + jnp.dot(ctx, wd_ref[...], precision=hi) + b2m_ref[...])
           + jnp.dot(Xm, wb_ref[...], precision=hi)
           + jnp.dot(Pm, wc_ref[...], precision=hi))
  out_ref[...] = (jnp.dot(xb, wax_ref[...], precision=hi)
                  + jnp.dot(aggr2, wag_ref[...], precision=hi) + b2a_ref[...])


def _tc_epilogue(acc2, x, pos, W1m, b1m, W1a, b1a, WA, WB, WC, WD, b2m,
                 WaX, WaG, b2a):
  BLK = 1000
  grid = (N_NODES // BLK,)
  full = lambda shape: pl.BlockSpec(shape, lambda b: (0,) * len(shape))
  return pl.pallas_call(
      _epilogue_body,
      grid=grid,
      in_specs=[
          pl.BlockSpec((NC, BLK, DG), lambda b: (0, b, 0)),
          pl.BlockSpec((BLK, D_FEAT), lambda b: (b, 0)),
          pl.BlockSpec((BLK, D_POS), lambda b: (b, 0)),
          full((D_POS, D_CTX)),
          full((1, D_CTX)),
          full((D_CTX, D_CTX)),
          full((1, D_CTX)),
          full((D_FEAT, D_FEAT)),
          full((D_FEAT, D_FEAT)),
          full((D_POS, D_FEAT)),
          full((D_CTX, D_FEAT)),
          full((1, D_FEAT)),
          full((D_FEAT, D_FEAT)),
          full((D_FEAT, D_FEAT)),
          full((1, D_FEAT)),
      ],
      out_specs=pl.BlockSpec((BLK, D_FEAT), lambda b: (b, 0)),
      out_shape=jax.ShapeDtypeStruct((N_NODES, D_FEAT), jnp.float32),
  )(acc2, x, pos, W1m, b1m, W1a, b1a, WA, WB, WC, WD, b2m, WaX, WaG, b2a)


def kernel(x, edge_index, pos, W1m, b1m, W1a, b1a, W2m, b2m, W2a, b2a):
  idx_i = edge_index[0].astype(jnp.int32)
  idx_j = edge_index[1].astype(jnp.int32)
  G = jnp.concatenate(
      [x, pos, jnp.ones((N_NODES, 1), jnp.float32),
       jnp.zeros((N_NODES, DG - D_FEAT - D_POS - 1), jnp.float32)], axis=1)
  zeros = jnp.zeros((NACC, DG), jnp.float32)
  acc2 = _sc_segment_sums(G, zeros, idx_i, idx_j)[:, :N_NODES]
  WA = W2m[:D_FEAT]
  WB = W2m[D_FEAT:2 * D_FEAT]
  WC = W2m[2 * D_FEAT:2 * D_FEAT + D_POS]
  WD = W2m[2 * D_FEAT + D_POS:]
  return _tc_epilogue(
      acc2, x, pos, W1m, b1m.reshape(1, -1), W1a, b1a.reshape(1, -1),
      WA, WB, WC, WD, b2m.reshape(1, -1), W2a[:D_FEAT], W2a[D_FEAT:],
      b2a.reshape(1, -1))


# speedup vs baseline: 13.1951x; 1.4041x over previous
import functools

import jax
import jax.numpy as jnp
from jax import lax
from jax.experimental import pallas as pl
from jax.experimental.pallas import tpu as pltpu
from jax.experimental.pallas import tpu_sc as plsc

N_NODES = 10000
N_EDGES = 320000
D_FEAT = 128
D_POS = 3
D_CTX = 32

NC = 2
NS = 16
NW = NC * NS
EPW = N_EDGES // NW
CH = 50
NCHUNK = EPW // CH
IDXB = 20
NIDXB = NCHUNK // IDXB
ISLOTS = 3
DG = 144
NACC = 10240
RPT = NACC // NS


NBUF = 4
LAG = 2


def _sc_segment_sums(G, zeros, idx_c):
  mesh = plsc.VectorSubcoreMesh(core_axis_name="c", subcore_axis_name="s")

  @functools.partial(
      pl.kernel,
      out_type=jax.ShapeDtypeStruct((NC, NACC, DG), jnp.float32),
      mesh=mesh,
      scratch_types=[
          pltpu.VMEM((ISLOTS, IDXB, 2, CH), jnp.int32),
          [pltpu.VMEM((CH, DG), jnp.float32)] * NBUF,
          pltpu.VMEM_SHARED((NACC, DG), jnp.float32),
          [pltpu.SemaphoreType.DMA] * NBUF,
          pltpu.SemaphoreType.DMA,
      ],
      compiler_params=pltpu.CompilerParams(use_tc_tiling_on_sc=False),
  )
  def k(g_hbm, z_hbm, idx_hbm, out_hbm, idxbuf, rows, acc, sems, isem):
    c = lax.axis_index("c")
    s = lax.axis_index("s")
    wid = c * NS + s
    pltpu.sync_copy(z_hbm.at[pl.ds(s * RPT, RPT)], acc.at[pl.ds(s * RPT, RPT)])
    plsc.subcore_barrier()

    def idxrow(cc, which):
      return idxbuf.at[(cc // IDXB) % ISLOTS, cc % IDXB, which]

    def gather_start(cc, b):
      pltpu.async_copy(g_hbm.at[idxrow(cc, 1)], rows[b], sems[b])

    def scatter_start(cc, b):
      pltpu.async_copy(rows[b], acc.at[idxrow(cc, 0)], sems[b], add=True)

    def drain(b):
      pltpu.make_async_copy(z_hbm.at[pl.ds(0, CH)], rows[b], sems[b]).wait()

    def stage_start(kb):
      pltpu.async_copy(idx_hbm.at[wid, kb], idxbuf.at[kb % ISLOTS], isem)

    def idrain():
      pltpu.make_async_copy(idx_hbm.at[0, 0], idxbuf.at[0], isem).wait()

    stage_start(0)

    def outer(kb, carry):
      idrain()

      @pl.when(kb + 1 < NIDXB)
      def _():
        stage_start(kb + 1)

      def inner(tt, carry2):
        for b in range(NBUF):
          cc = kb * IDXB + tt * NBUF + b
          @pl.when(cc >= NBUF)
          def _():
            drain(b)
          gather_start(cc, b)
          c2 = cc - LAG
          b2 = (b - LAG) % NBUF

          @pl.when(c2 >= 0)
          def _():
            drain(b2)
            scatter_start(c2, b2)
        return carry2

      return lax.fori_loop(0, IDXB // NBUF, inner, carry)

    lax.fori_loop(0, NIDXB, outer, 0)
    for c2 in range(NCHUNK - LAG, NCHUNK):
      b2 = c2 % NBUF
      drain(b2)
      scatter_start(c2, b2)
    for b in range(NBUF):
      drain(b)
    plsc.subcore_barrier()
    pltpu.sync_copy(acc.at[pl.ds(s * RPT, RPT)],
                    out_hbm.at[c, pl.ds(s * RPT, RPT)])

  return k(G, zeros, idx_c)


def _epilogue_body(acc_ref, x_ref, pos_ref, w1m_ref, b1m_ref, w1a_ref,
                   b1a_ref, wa_ref, wb_ref, wc_ref, wd_ref, b2m_ref,
                   wax_ref, wag_ref, b2a_ref, out_ref):
  a = acc_ref[0] + acc_ref[1]
  Sx = a[:, :D_FEAT]
  Sp = a[:, D_FEAT:D_FEAT + D_POS]
  cnt = a[:, D_FEAT + D_POS:D_FEAT + D_POS + 1]
  has = (cnt > 0.0).astype(jnp.float32)
  inv = 1.0 / jnp.maximum(cnt, 1.0)
  xb = x_ref[...]
  Xm = Sx * inv
  Pm = Sp * inv - has * pos_ref[...]
  hi = lax.Precision.HIGHEST
  aggr1 = jnp.dot(Pm, w1m_ref[...], precision=hi) + has * b1m_ref[...]
  ctx = jnp.dot(aggr1, w1a_ref[...], precision=hi) + b1a_ref[...]
  aggr2 = (has * (jnp.dot(xb, wa_ref[...] - wb_ref[...], precision=hi)
                  + jnp.dot(ctx, wd_ref[...], precision=hi) + b2m_ref[...])
           + jnp.dot(Xm, wb_ref[...], precision=hi)
           + jnp.dot(Pm, wc_ref[...], precision=hi))
  out_ref[...] = (jnp.dot(xb, wax_ref[...], precision=hi)
                  + jnp.dot(aggr2, wag_ref[...], precision=hi) + b2a_ref[...])


def _tc_epilogue(acc2, x, pos, W1m, b1m, W1a, b1a, WA, WB, WC, WD, b2m,
                 WaX, WaG, b2a):
  BLK = 1000
  grid = (N_NODES // BLK,)
  full = lambda shape: pl.BlockSpec(shape, lambda b: (0,) * len(shape))
  return pl.pallas_call(
      _epilogue_body,
      grid=grid,
      in_specs=[
          pl.BlockSpec((NC, BLK, DG), lambda b: (0, b, 0)),
          pl.BlockSpec((BLK, D_FEAT), lambda b: (b, 0)),
          pl.BlockSpec((BLK, D_POS), lambda b: (b, 0)),
          full((D_POS, D_CTX)),
          full((1, D_CTX)),
          full((D_CTX, D_CTX)),
          full((1, D_CTX)),
          full((D_FEAT, D_FEAT)),
          full((D_FEAT, D_FEAT)),
          full((D_POS, D_FEAT)),
          full((D_CTX, D_FEAT)),
          full((1, D_FEAT)),
          full((D_FEAT, D_FEAT)),
          full((D_FEAT, D_FEAT)),
          full((1, D_FEAT)),
      ],
      out_specs=pl.BlockSpec((BLK, D_FEAT), lambda b: (b, 0)),
      out_shape=jax.ShapeDtypeStruct((N_NODES, D_FEAT), jnp.float32),
  )(acc2, x, pos, W1m, b1m, W1a, b1a, WA, WB, WC, WD, b2m, WaX, WaG, b2a)


def kernel(x, edge_index, pos, W1m, b1m, W1a, b1a, W2m, b2m, W2a, b2a):
  ei = edge_index.astype(jnp.int32)
  idx_c = jnp.stack(
      [ei[0].reshape(NW, NCHUNK, CH), ei[1].reshape(NW, NCHUNK, CH)],
      axis=2).reshape(NW, NIDXB, IDXB, 2, CH)
  G = jnp.concatenate(
      [x, pos, jnp.ones((N_NODES, 1), jnp.float32),
       jnp.zeros((N_NODES, DG - D_FEAT - D_POS - 1), jnp.float32)], axis=1)
  zeros = jnp.zeros((NACC, DG), jnp.float32)
  acc2 = _sc_segment_sums(G, zeros, idx_c)[:, :N_NODES]
  WA = W2m[:D_FEAT]
  WB = W2m[D_FEAT:2 * D_FEAT]
  WC = W2m[2 * D_FEAT:2 * D_FEAT + D_POS]
  WD = W2m[2 * D_FEAT + D_POS:]
  return _tc_epilogue(
      acc2, x, pos, W1m, b1m.reshape(1, -1), W1a, b1a.reshape(1, -1),
      WA, WB, WC, WD, b2m.reshape(1, -1), W2a[:D_FEAT], W2a[D_FEAT:],
      b2a.reshape(1, -1))


# speedup vs baseline: 21.8440x; 1.6555x over previous
import functools

import jax
import jax.numpy as jnp
from jax import lax
from jax.experimental import pallas as pl
from jax.experimental.pallas import tpu as pltpu
from jax.experimental.pallas import tpu_sc as plsc

N_NODES = 10000
N_EDGES = 320000
D_FEAT = 128
D_POS = 3
D_CTX = 32

NC = 2
NS = 16
NW = NC * NS
EPW = N_EDGES // NW
CH = 50
NCHUNK = EPW // CH
IDXB = 20
NIDXB = NCHUNK // IDXB
ISLOTS = 3
DG = 144
DP = DG - D_FEAT
NACC = 10240
RPT = NACC // NS

NBUF = 4
LAG = 2

BLK = 1000
KRHS = 3 * D_FEAT + D_CTX + DP


def _sc_segment_sums(G, zeros, idx5):
  mesh = plsc.VectorSubcoreMesh(core_axis_name="c", subcore_axis_name="s")

  @functools.partial(
      pl.kernel,
      out_type=jax.ShapeDtypeStruct((NC, NACC, DG), jnp.float32),
      mesh=mesh,
      scratch_types=[
          pltpu.VMEM((ISLOTS, 2, IDXB, CH), jnp.int32),
          [pltpu.VMEM((CH, DG), jnp.float32)] * NBUF,
          pltpu.VMEM_SHARED((NACC, DG), jnp.float32),
          [pltpu.SemaphoreType.DMA] * NBUF,
          pltpu.SemaphoreType.DMA,
      ],
      compiler_params=pltpu.CompilerParams(use_tc_tiling_on_sc=False),
  )
  def k(g_hbm, z_hbm, idx_hbm, out_hbm, idxbuf, rows, acc, sems, isem):
    c = lax.axis_index("c")
    s = lax.axis_index("s")
    wid = c * NS + s
    pltpu.sync_copy(z_hbm.at[pl.ds(s * RPT, RPT)], acc.at[pl.ds(s * RPT, RPT)])
    plsc.subcore_barrier()

    def idxrow(cc, which):
      return idxbuf.at[(cc // IDXB) % ISLOTS, which, cc % IDXB]

    def gather_start(cc, b):
      pltpu.async_copy(g_hbm.at[idxrow(cc, 1)], rows[b], sems[b])

    def scatter_start(cc, b):
      pltpu.async_copy(rows[b], acc.at[idxrow(cc, 0)], sems[b], add=True)

    def drain(b):
      pltpu.make_async_copy(z_hbm.at[pl.ds(0, CH)], rows[b], sems[b]).wait()

    def stage_start(kb):
      sl = kb % ISLOTS
      pltpu.async_copy(idx_hbm.at[0, wid, kb], idxbuf.at[sl, 0], isem)
      pltpu.async_copy(idx_hbm.at[1, wid, kb], idxbuf.at[sl, 1], isem)

    def idrain():
      for _ in range(2):
        pltpu.make_async_copy(idx_hbm.at[0, 0, 0], idxbuf.at[0, 0],
                              isem).wait()

    stage_start(0)

    def outer(kb, carry):
      idrain()

      @pl.when(kb + 1 < NIDXB)
      def _():
        stage_start(kb + 1)

      def inner(tt, carry2):
        for b in range(NBUF):
          cc = kb * IDXB + tt * NBUF + b
          @pl.when(cc >= NBUF)
          def _():
            drain(b)
          gather_start(cc, b)
          c2 = cc - LAG
          b2 = (b - LAG) % NBUF

          @pl.when(c2 >= 0)
          def _():
            drain(b2)
            scatter_start(c2, b2)
        return carry2

      return lax.fori_loop(0, IDXB // NBUF, inner, carry)

    lax.fori_loop(0, NIDXB, outer, 0)
    for c2 in range(NCHUNK - LAG, NCHUNK):
      b2 = c2 % NBUF
      drain(b2)
      scatter_start(c2, b2)
    for b in range(NBUF):
      drain(b)
    plsc.subcore_barrier()
    pltpu.sync_copy(acc.at[pl.ds(s * RPT, RPT)],
                    out_hbm.at[c, pl.ds(s * RPT, RPT)])

  return k(G, zeros, idx5)


def _builder_body(x_ref, pos_ref, wa_ref, wb_ref, wc_ref, wd_ref, wax_ref,
                  wag_ref, b2m_ref, g_ref, p16_ref, rhs_ref, f5_ref):
  g_ref[:, :D_FEAT] = x_ref[...]
  p16 = jnp.concatenate(
      [pos_ref[...], jnp.ones((BLK, 1), jnp.float32),
       jnp.zeros((BLK, DP - D_POS - 1), jnp.float32)], axis=1)
  g_ref[:, D_FEAT:] = p16
  p16_ref[...] = p16

  @pl.when(pl.program_id(0) == 0)
  def _():
    hi = lax.Precision.HIGHEST
    wag = wag_ref[...]
    rhs_ref[0:D_FEAT, :] = jnp.dot(wa_ref[...] - wb_ref[...], wag,
                                   precision=hi)
    rhs_ref[D_FEAT:D_FEAT + D_CTX, :] = jnp.dot(wd_ref[...], wag,
                                                precision=hi)
    rhs_ref[D_FEAT + D_CTX:2 * D_FEAT + D_CTX, :] = jnp.dot(
        wb_ref[...], wag, precision=hi)
    c16 = jnp.concatenate(
        [wc_ref[...], jnp.zeros((DP - D_POS, D_FEAT), jnp.float32)], axis=0)
    rhs_ref[2 * D_FEAT + D_CTX:2 * D_FEAT + D_CTX + DP, :] = jnp.dot(
        c16, wag, precision=hi)
    rhs_ref[2 * D_FEAT + D_CTX + DP:, :] = wax_ref[...]
    f5_ref[...] = jnp.dot(b2m_ref[...], wag, precision=hi)


def _tc_builder(x, pos, WA, WB, WC, WD, WaX, WaG, b2m):
  full = lambda shape: pl.BlockSpec(shape, lambda b: (0,) * len(shape))
  return pl.pallas_call(
      _builder_body,
      grid=(N_NODES // BLK,),
      in_specs=[
          pl.BlockSpec((BLK, D_FEAT), lambda b: (b, 0)),
          pl.BlockSpec((BLK, D_POS), lambda b: (b, 0)),
          full((D_FEAT, D_FEAT)),
          full((D_FEAT, D_FEAT)),
          full((D_POS, D_FEAT)),
          full((D_CTX, D_FEAT)),
          full((D_FEAT, D_FEAT)),
          full((D_FEAT, D_FEAT)),
          full((1, D_FEAT)),
      ],
      out_specs=[
          pl.BlockSpec((BLK, DG), lambda b: (b, 0)),
          pl.BlockSpec((BLK, DP), lambda b: (b, 0)),
          full((KRHS, D_FEAT)),
          full((1, D_FEAT)),
      ],
      out_shape=[
          jax.ShapeDtypeStruct((N_NODES, DG), jnp.float32),
          jax.ShapeDtypeStruct((N_NODES, DP), jnp.float32),
          jax.ShapeDtypeStruct((KRHS, D_FEAT), jnp.float32),
          jax.ShapeDtypeStruct((1, D_FEAT), jnp.float32),
      ],
  )(x, pos, WA, WB, WC, WD, WaX, WaG, b2m)


def _epilogue_body(acc_ref, x_ref, gp_ref, w1m_ref, b1m_ref, w1a_ref,
                   b1a_ref, rhs_ref, f5_ref, b2a_ref, out_ref):
  a = acc_ref[0] + acc_ref[1]
  Sx = a[:, :D_FEAT]
  Sp16 = a[:, D_FEAT:]
  cnt = a[:, D_FEAT + D_POS:D_FEAT + D_POS + 1]
  has = (cnt > 0.0).astype(jnp.float32)
  inv = 1.0 / jnp.maximum(cnt, 1.0)
  xb = x_ref[...]
  Xm = Sx * inv
  Pm16 = Sp16 * inv - has * gp_ref[...]
  ctx = jnp.dot(jnp.dot(Pm16, w1m_ref[...]) + has * b1m_ref[...],
                w1a_ref[...]) + b1a_ref[...]
  M = jnp.concatenate([has * xb, has * ctx, Xm, Pm16, xb], axis=1)
  out_ref[...] = (jnp.dot(M, rhs_ref[...]) + has * f5_ref[...]
                  + b2a_ref[...])


def _tc_epilogue(acc2, x, G, W1m16, b1m, W1a, b1a, RHS, F5, b2a):
  full = lambda shape: pl.BlockSpec(shape, lambda b: (0,) * len(shape))
  return pl.pallas_call(
      _epilogue_body,
      grid=(N_NODES // BLK,),
      in_specs=[
          pl.BlockSpec((NC, BLK, DG), lambda b: (0, b, 0)),
          pl.BlockSpec((BLK, D_FEAT), lambda b: (b, 0)),
          pl.BlockSpec((BLK, DP), lambda b: (b, 0)),
          full((DP, D_CTX)),
          full((1, D_CTX)),
          full((D_CTX, D_CTX)),
          full((1, D_CTX)),
          full((KRHS, D_FEAT)),
          full((1, D_FEAT)),
          full((1, D_FEAT)),
      ],
      out_specs=pl.BlockSpec((BLK, D_FEAT), lambda b: (b, 0)),
      out_shape=jax.ShapeDtypeStruct((N_NODES, D_FEAT), jnp.float32),
  )(acc2, x, G, W1m16, b1m, W1a, b1a, RHS, F5, b2a)


def kernel(x, edge_index, pos, W1m, b1m, W1a, b1a, W2m, b2m, W2a, b2a):
  idx5 = edge_index.astype(jnp.int32).reshape(2, NW, NIDXB, IDXB, CH)
  zeros = jnp.zeros((NACC, DG), jnp.float32)
  WA = W2m[:D_FEAT]
  WB = W2m[D_FEAT:2 * D_FEAT]
  WC = W2m[2 * D_FEAT:2 * D_FEAT + D_POS]
  WD = W2m[2 * D_FEAT + D_POS:]
  G, P16, RHS, F5 = _tc_builder(x, pos, WA, WB, WC, WD, W2a[:D_FEAT],
                                W2a[D_FEAT:], b2m.reshape(1, -1))
  acc2 = _sc_segment_sums(G, zeros, idx5)
  W1m16 = jnp.pad(W1m, ((0, DP - D_POS), (0, 0)))
  return _tc_epilogue(acc2, x, P16, W1m16, b1m.reshape(1, -1), W1a,
                      b1a.reshape(1, -1), RHS, F5, b2a.reshape(1, -1))


# speedup vs baseline: 23.0414x; 1.0548x over previous
import functools

import jax
import jax.numpy as jnp
from jax import lax
from jax.experimental import pallas as pl
from jax.experimental.pallas import tpu as pltpu
from jax.experimental.pallas import tpu_sc as plsc

N_NODES = 10000
N_EDGES = 320000
D_FEAT = 128
D_POS = 3
D_CTX = 32

NC = 2
NS = 16
NW = NC * NS
EPW = N_EDGES // NW
CH = 50
NCHUNK = EPW // CH
IDXB = 20
NIDXB = NCHUNK // IDXB
ISLOTS = 3
DG = 144
DP = DG - D_FEAT
NACC = 10240
RPT = NACC // NS

NBUF = 4
LAG = 2

BLK = 1000
KRHS = 3 * D_FEAT + D_CTX + DP


def _sc_segment_sums(G, zeros, idx5):
  mesh = plsc.VectorSubcoreMesh(core_axis_name="c", subcore_axis_name="s")

  @functools.partial(
      pl.kernel,
      out_type=[jax.ShapeDtypeStruct((NC, NACC, D_FEAT), jnp.float32),
                jax.ShapeDtypeStruct((NC, NACC, DP), jnp.float32)],
      mesh=mesh,
      scratch_types=[
          pltpu.VMEM((ISLOTS, 2, IDXB, CH), jnp.int32),
          [pltpu.VMEM((CH, DG), jnp.float32)] * NBUF,
          pltpu.VMEM_SHARED((NACC, DG), jnp.float32),
          [pltpu.SemaphoreType.DMA] * NBUF,
          pltpu.SemaphoreType.DMA,
      ],
      compiler_params=pltpu.CompilerParams(use_tc_tiling_on_sc=False),
  )
  def k(g_hbm, z_hbm, idx_hbm, outx_hbm, outp_hbm, idxbuf, rows, acc, sems,
        isem):
    c = lax.axis_index("c")
    s = lax.axis_index("s")
    wid = c * NS + s
    pltpu.sync_copy(z_hbm.at[pl.ds(s * RPT, RPT)], acc.at[pl.ds(s * RPT, RPT)])
    plsc.subcore_barrier()

    def idxrow(cc, which):
      return idxbuf.at[(cc // IDXB) % ISLOTS, which, cc % IDXB]

    def gather_start(cc, b):
      pltpu.async_copy(g_hbm.at[idxrow(cc, 1)], rows[b], sems[b])

    def scatter_start(cc, b):
      pltpu.async_copy(rows[b], acc.at[idxrow(cc, 0)], sems[b], add=True)

    def drain(b):
      pltpu.make_async_copy(z_hbm.at[pl.ds(0, CH)], rows[b], sems[b]).wait()

    def stage_start(kb):
      sl = kb % ISLOTS
      pltpu.async_copy(idx_hbm.at[0, wid, kb], idxbuf.at[sl, 0], isem)
      pltpu.async_copy(idx_hbm.at[1, wid, kb], idxbuf.at[sl, 1], isem)

    def idrain():
      for _ in range(2):
        pltpu.make_async_copy(idx_hbm.at[0, 0, 0], idxbuf.at[0, 0],
                              isem).wait()

    stage_start(0)

    def outer(kb, carry):
      idrain()

      @pl.when(kb + 1 < NIDXB)
      def _():
        stage_start(kb + 1)

      def inner(tt, carry2):
        for b in range(NBUF):
          cc = kb * IDXB + tt * NBUF + b
          @pl.when(cc >= NBUF)
          def _():
            drain(b)
          gather_start(cc, b)
          c2 = cc - LAG
          b2 = (b - LAG) % NBUF

          @pl.when(c2 >= 0)
          def _():
            drain(b2)
            scatter_start(c2, b2)
        return carry2

      return lax.fori_loop(0, IDXB // NBUF, inner, carry)

    lax.fori_loop(0, NIDXB, outer, 0)
    for c2 in range(NCHUNK - LAG, NCHUNK):
      b2 = c2 % NBUF
      drain(b2)
      scatter_start(c2, b2)
    for b in range(NBUF):
      drain(b)
    plsc.subcore_barrier()
    pltpu.sync_copy(acc.at[pl.ds(s * RPT, RPT), pl.ds(0, D_FEAT)],
                    outx_hbm.at[c, pl.ds(s * RPT, RPT)])
    pltpu.sync_copy(acc.at[pl.ds(s * RPT, RPT), pl.ds(D_FEAT, DP)],
                    outp_hbm.at[c, pl.ds(s * RPT, RPT)])

  return k(G, zeros, idx5)


def _builder_body(x_ref, pos_ref, wa_ref, wb_ref, wc_ref, wd_ref, wax_ref,
                  wag_ref, b2m_ref, g_ref, p16_ref, rhs_ref, f5_ref):
  g_ref[:, :D_FEAT] = x_ref[...]
  p16 = jnp.concatenate(
      [pos_ref[...], jnp.ones((BLK, 1), jnp.float32),
       jnp.zeros((BLK, DP - D_POS - 1), jnp.float32)], axis=1)
  g_ref[:, D_FEAT:] = p16
  p16_ref[...] = p16

  @pl.when(pl.program_id(0) == 0)
  def _():
    hi = lax.Precision.HIGHEST
    wag = wag_ref[...]
    rhs_ref[0:D_FEAT, :] = jnp.dot(wa_ref[...] - wb_ref[...], wag,
                                   precision=hi)
    rhs_ref[D_FEAT:D_FEAT + D_CTX, :] = jnp.dot(wd_ref[...], wag,
                                                precision=hi)
    rhs_ref[D_FEAT + D_CTX:2 * D_FEAT + D_CTX, :] = jnp.dot(
        wb_ref[...], wag, precision=hi)
    c16 = jnp.concatenate(
        [wc_ref[...], jnp.zeros((DP - D_POS, D_FEAT), jnp.float32)], axis=0)
    rhs_ref[2 * D_FEAT + D_CTX:2 * D_FEAT + D_CTX + DP, :] = jnp.dot(
        c16, wag, precision=hi)
    rhs_ref[2 * D_FEAT + D_CTX + DP:, :] = wax_ref[...]
    f5_ref[...] = jnp.dot(b2m_ref[...], wag, precision=hi)


def _tc_builder(x, pos, WA, WB, WC, WD, WaX, WaG, b2m):
  full = lambda shape: pl.BlockSpec(shape, lambda b: (0,) * len(shape))
  return pl.pallas_call(
      _builder_body,
      grid=(N_NODES // BLK,),
      in_specs=[
          pl.BlockSpec((BLK, D_FEAT), lambda b: (b, 0)),
          pl.BlockSpec((BLK, D_POS), lambda b: (b, 0)),
          full((D_FEAT, D_FEAT)),
          full((D_FEAT, D_FEAT)),
          full((D_POS, D_FEAT)),
          full((D_CTX, D_FEAT)),
          full((D_FEAT, D_FEAT)),
          full((D_FEAT, D_FEAT)),
          full((1, D_FEAT)),
      ],
      out_specs=[
          pl.BlockSpec((BLK, DG), lambda b: (b, 0)),
          pl.BlockSpec((BLK, DP), lambda b: (b, 0)),
          full((KRHS, D_FEAT)),
          full((1, D_FEAT)),
      ],
      out_shape=[
          jax.ShapeDtypeStruct((N_NODES, DG), jnp.float32),
          jax.ShapeDtypeStruct((N_NODES, DP), jnp.float32),
          jax.ShapeDtypeStruct((KRHS, D_FEAT), jnp.float32),
          jax.ShapeDtypeStruct((1, D_FEAT), jnp.float32),
      ],
  )(x, pos, WA, WB, WC, WD, WaX, WaG, b2m)


def _epilogue_body(accx_ref, accp_ref, x_ref, gp_ref, w1m_ref, b1m_ref,
                   w1a_ref, b1a_ref, rhs_ref, f5_ref, b2a_ref, out_ref):
  Sx = accx_ref[0] + accx_ref[1]
  Sp16 = accp_ref[0] + accp_ref[1]
  cnt = Sp16[:, D_POS:D_POS + 1]
  has = (cnt > 0.0).astype(jnp.float32)
  inv = 1.0 / jnp.maximum(cnt, 1.0)
  xb = x_ref[...]
  Xm = Sx * inv
  Pm16 = Sp16 * inv - has * gp_ref[...]
  ctx = jnp.dot(jnp.dot(Pm16, w1m_ref[...]) + has * b1m_ref[...],
                w1a_ref[...]) + b1a_ref[...]
  M = jnp.concatenate([has * xb, has * ctx, Xm, Pm16, xb], axis=1)
  out_ref[...] = (jnp.dot(M, rhs_ref[...]) + has * f5_ref[...]
                  + b2a_ref[...])


def _tc_epilogue(accx, accp, x, P16, W1m16, b1m, W1a, b1a, RHS, F5, b2a):
  full = lambda shape: pl.BlockSpec(shape, lambda b: (0,) * len(shape))
  return pl.pallas_call(
      _epilogue_body,
      grid=(N_NODES // BLK,),
      in_specs=[
          pl.BlockSpec((NC, BLK, D_FEAT), lambda b: (0, b, 0)),
          pl.BlockSpec((NC, BLK, DP), lambda b: (0, b, 0)),
          pl.BlockSpec((BLK, D_FEAT), lambda b: (b, 0)),
          pl.BlockSpec((BLK, DP), lambda b: (b, 0)),
          full((DP, D_CTX)),
          full((1, D_CTX)),
          full((D_CTX, D_CTX)),
          full((1, D_CTX)),
          full((KRHS, D_FEAT)),
          full((1, D_FEAT)),
          full((1, D_FEAT)),
      ],
      out_specs=pl.BlockSpec((BLK, D_FEAT), lambda b: (b, 0)),
      out_shape=jax.ShapeDtypeStruct((N_NODES, D_FEAT), jnp.float32),
  )(accx, accp, x, P16, W1m16, b1m, W1a, b1a, RHS, F5, b2a)


def kernel(x, edge_index, pos, W1m, b1m, W1a, b1a, W2m, b2m, W2a, b2a):
  idx5 = edge_index.astype(jnp.int32).reshape(2, NW, NIDXB, IDXB, CH)
  zeros = jnp.zeros((NACC, DG), jnp.float32)
  WA = W2m[:D_FEAT]
  WB = W2m[D_FEAT:2 * D_FEAT]
  WC = W2m[2 * D_FEAT:2 * D_FEAT + D_POS]
  WD = W2m[2 * D_FEAT + D_POS:]
  G, P16, RHS, F5 = _tc_builder(x, pos, WA, WB, WC, WD, W2a[:D_FEAT],
                                W2a[D_FEAT:], b2m.reshape(1, -1))
  accx, accp = _sc_segment_sums(G, zeros, idx5)
  W1m16 = jnp.pad(W1m, ((0, DP - D_POS), (0, 0)))
  return _tc_epilogue(accx, accp, x, P16, W1m16, b1m.reshape(1, -1), W1a,
                      b1a.reshape(1, -1), RHS, F5, b2a.reshape(1, -1))


# speedup vs baseline: 23.5291x; 1.0212x over previous
import functools

import jax
import jax.numpy as jnp
from jax import lax
from jax.experimental import pallas as pl
from jax.experimental.pallas import tpu as pltpu
from jax.experimental.pallas import tpu_sc as plsc

N_NODES = 10000
N_EDGES = 320000
D_FEAT = 128
D_POS = 3
D_CTX = 32

NC = 2
NS = 16
NW = NC * NS
EPW = N_EDGES // NW
CH = 100
NCHUNK = EPW // CH
IDXB = 10
NIDXB = NCHUNK // IDXB
ISLOTS = 3
DG = 144
DP = DG - D_FEAT
NACC = 10240
RPT = NACC // NS

NBUF = 2
LAG = 1

BLK = 1000
KRHS = 3 * D_FEAT + D_CTX + DP


def _sc_segment_sums(G, zeros, idx5):
  mesh = plsc.VectorSubcoreMesh(core_axis_name="c", subcore_axis_name="s")

  @functools.partial(
      pl.kernel,
      out_type=[jax.ShapeDtypeStruct((NC, NACC, D_FEAT), jnp.float32),
                jax.ShapeDtypeStruct((NC, NACC, DP), jnp.float32)],
      mesh=mesh,
      scratch_types=[
          pltpu.VMEM((ISLOTS, 2, IDXB, CH), jnp.int32),
          [pltpu.VMEM((CH, DG), jnp.float32)] * NBUF,
          pltpu.VMEM_SHARED((NACC, DG), jnp.float32),
          [pltpu.SemaphoreType.DMA] * NBUF,
          pltpu.SemaphoreType.DMA,
      ],
      compiler_params=pltpu.CompilerParams(use_tc_tiling_on_sc=False),
  )
  def k(g_hbm, z_hbm, idx_hbm, outx_hbm, outp_hbm, idxbuf, rows, acc, sems,
        isem):
    c = lax.axis_index("c")
    s = lax.axis_index("s")
    wid = c * NS + s
    pltpu.sync_copy(z_hbm.at[pl.ds(s * RPT, RPT)], acc.at[pl.ds(s * RPT, RPT)])
    plsc.subcore_barrier()

    def idxrow(cc, which):
      return idxbuf.at[(cc // IDXB) % ISLOTS, which, cc % IDXB]

    def gather_start(cc, b):
      pltpu.async_copy(g_hbm.at[idxrow(cc, 1)], rows[b], sems[b])

    def scatter_start(cc, b):
      pltpu.async_copy(rows[b], acc.at[idxrow(cc, 0)], sems[b], add=True)

    def drain(b):
      pltpu.make_async_copy(z_hbm.at[pl.ds(0, CH)], rows[b], sems[b]).wait()

    def stage_start(kb):
      sl = kb % ISLOTS
      pltpu.async_copy(idx_hbm.at[0, wid, kb], idxbuf.at[sl, 0], isem)
      pltpu.async_copy(idx_hbm.at[1, wid, kb], idxbuf.at[sl, 1], isem)

    def idrain():
      for _ in range(2):
        pltpu.make_async_copy(idx_hbm.at[0, 0, 0], idxbuf.at[0, 0],
                              isem).wait()

    stage_start(0)

    def outer(kb, carry):
      idrain()

      @pl.when(kb + 1 < NIDXB)
      def _():
        stage_start(kb + 1)

      def inner(tt, carry2):
        for b in range(NBUF):
          cc = kb * IDXB + tt * NBUF + b
          @pl.when(cc >= NBUF)
          def _():
            drain(b)
          gather_start(cc, b)
          c2 = cc - LAG
          b2 = (b - LAG) % NBUF

          @pl.when(c2 >= 0)
          def _():
            drain(b2)
            scatter_start(c2, b2)
        return carry2

      return lax.fori_loop(0, IDXB // NBUF, inner, carry)

    lax.fori_loop(0, NIDXB, outer, 0)
    for c2 in range(NCHUNK - LAG, NCHUNK):
      b2 = c2 % NBUF
      drain(b2)
      scatter_start(c2, b2)
    for b in range(NBUF):
      drain(b)
    plsc.subcore_barrier()
    pltpu.sync_copy(acc.at[pl.ds(s * RPT, RPT), pl.ds(0, D_FEAT)],
                    outx_hbm.at[c, pl.ds(s * RPT, RPT)])
    pltpu.sync_copy(acc.at[pl.ds(s * RPT, RPT), pl.ds(D_FEAT, DP)],
                    outp_hbm.at[c, pl.ds(s * RPT, RPT)])

  return k(G, zeros, idx5)


def _builder_body(x_ref, pos_ref, wa_ref, wb_ref, wc_ref, wd_ref, wax_ref,
                  wag_ref, b2m_ref, g_ref, p16_ref, rhs_ref, f5_ref):
  g_ref[:, :D_FEAT] = x_ref[...]
  p16 = jnp.concatenate(
      [pos_ref[...], jnp.ones((BLK, 1), jnp.float32),
       jnp.zeros((BLK, DP - D_POS - 1), jnp.float32)], axis=1)
  g_ref[:, D_FEAT:] = p16
  p16_ref[...] = p16

  @pl.when(pl.program_id(0) == 0)
  def _():
    hi = lax.Precision.HIGHEST
    wag = wag_ref[...]
    rhs_ref[0:D_FEAT, :] = jnp.dot(wa_ref[...] - wb_ref[...], wag,
                                   precision=hi)
    rhs_ref[D_FEAT:D_FEAT + D_CTX, :] = jnp.dot(wd_ref[...], wag,
                                                precision=hi)
    rhs_ref[D_FEAT + D_CTX:2 * D_FEAT + D_CTX, :] = jnp.dot(
        wb_ref[...], wag, precision=hi)
    c16 = jnp.concatenate(
        [wc_ref[...], jnp.zeros((DP - D_POS, D_FEAT), jnp.float32)], axis=0)
    rhs_ref[2 * D_FEAT + D_CTX:2 * D_FEAT + D_CTX + DP, :] = jnp.dot(
        c16, wag, precision=hi)
    rhs_ref[2 * D_FEAT + D_CTX + DP:, :] = wax_ref[...]
    f5_ref[...] = jnp.dot(b2m_ref[...], wag, precision=hi)


def _tc_builder(x, pos, WA, WB, WC, WD, WaX, WaG, b2m):
  full = lambda shape: pl.BlockSpec(shape, lambda b: (0,) * len(shape))
  return pl.pallas_call(
      _builder_body,
      grid=(N_NODES // BLK,),
      in_specs=[
          pl.BlockSpec((BLK, D_FEAT), lambda b: (b, 0)),
          pl.BlockSpec((BLK, D_POS), lambda b: (b, 0)),
          full((D_FEAT, D_FEAT)),
          full((D_FEAT, D_FEAT)),
          full((D_POS, D_FEAT)),
          full((D_CTX, D_FEAT)),
          full((D_FEAT, D_FEAT)),
          full((D_FEAT, D_FEAT)),
          full((1, D_FEAT)),
      ],
      out_specs=[
          pl.BlockSpec((BLK, DG), lambda b: (b, 0)),
          pl.BlockSpec((BLK, DP), lambda b: (b, 0)),
          full((KRHS, D_FEAT)),
          full((1, D_FEAT)),
      ],
      out_shape=[
          jax.ShapeDtypeStruct((N_NODES, DG), jnp.float32),
          jax.ShapeDtypeStruct((N_NODES, DP), jnp.float32),
          jax.ShapeDtypeStruct((KRHS, D_FEAT), jnp.float32),
          jax.ShapeDtypeStruct((1, D_FEAT), jnp.float32),
      ],
  )(x, pos, WA, WB, WC, WD, WaX, WaG, b2m)


def _epilogue_body(accx_ref, accp_ref, x_ref, gp_ref, w1m_ref, b1m_ref,
                   w1a_ref, b1a_ref, rhs_ref, f5_ref, b2a_ref, out_ref):
  Sx = accx_ref[0] + accx_ref[1]
  Sp16 = accp_ref[0] + accp_ref[1]
  cnt = Sp16[:, D_POS:D_POS + 1]
  has = (cnt > 0.0).astype(jnp.float32)
  inv = 1.0 / jnp.maximum(cnt, 1.0)
  xb = x_ref[...]
  Xm = Sx * inv
  Pm16 = Sp16 * inv - has * gp_ref[...]
  ctx = jnp.dot(jnp.dot(Pm16, w1m_ref[...]) + has * b1m_ref[...],
                w1a_ref[...]) + b1a_ref[...]
  M = jnp.concatenate([has * xb, has * ctx, Xm, Pm16, xb], axis=1)
  out_ref[...] = (jnp.dot(M, rhs_ref[...]) + has * f5_ref[...]
                  + b2a_ref[...])


def _tc_epilogue(accx, accp, x, P16, W1m16, b1m, W1a, b1a, RHS, F5, b2a):
  full = lambda shape: pl.BlockSpec(shape, lambda b: (0,) * len(shape))
  return pl.pallas_call(
      _epilogue_body,
      grid=(N_NODES // BLK,),
      in_specs=[
          pl.BlockSpec((NC, BLK, D_FEAT), lambda b: (0, b, 0)),
          pl.BlockSpec((NC, BLK, DP), lambda b: (0, b, 0)),
          pl.BlockSpec((BLK, D_FEAT), lambda b: (b, 0)),
          pl.BlockSpec((BLK, DP), lambda b: (b, 0)),
          full((DP, D_CTX)),
          full((1, D_CTX)),
          full((D_CTX, D_CTX)),
          full((1, D_CTX)),
          full((KRHS, D_FEAT)),
          full((1, D_FEAT)),
          full((1, D_FEAT)),
      ],
      out_specs=pl.BlockSpec((BLK, D_FEAT), lambda b: (b, 0)),
      out_shape=jax.ShapeDtypeStruct((N_NODES, D_FEAT), jnp.float32),
  )(accx, accp, x, P16, W1m16, b1m, W1a, b1a, RHS, F5, b2a)


def kernel(x, edge_index, pos, W1m, b1m, W1a, b1a, W2m, b2m, W2a, b2a):
  idx5 = edge_index.astype(jnp.int32).reshape(2, NW, NIDXB, IDXB, CH)
  zeros = jnp.zeros((NACC, DG), jnp.float32)
  WA = W2m[:D_FEAT]
  WB = W2m[D_FEAT:2 * D_FEAT]
  WC = W2m[2 * D_FEAT:2 * D_FEAT + D_POS]
  WD = W2m[2 * D_FEAT + D_POS:]
  G, P16, RHS, F5 = _tc_builder(x, pos, WA, WB, WC, WD, W2a[:D_FEAT],
                                W2a[D_FEAT:], b2m.reshape(1, -1))
  accx, accp = _sc_segment_sums(G, zeros, idx5)
  W1m16 = jnp.pad(W1m, ((0, DP - D_POS), (0, 0)))
  return _tc_epilogue(accx, accp, x, P16, W1m16, b1m.reshape(1, -1), W1a,
                      b1a.reshape(1, -1), RHS, F5, b2a.reshape(1, -1))
